# Initial kernel scaffold; baseline (speedup 1.0000x reference)
import jax
import jax.numpy as jnp
from jax.experimental import pallas as pl


def kernel(query_points, ref_points, ref_flow):
    raise NotImplementedError("write your pallas kernel here")


# speedup vs baseline: 4.0151x; 4.0151x over previous
import functools

import jax
import jax.numpy as jnp
from jax.experimental import pallas as pl

QB = 256
M = 16384
DPAD = 8
K = 3


def _body(q_ref, rT_ref, flow_ref, out_ref):
    q = q_ref[...]
    rT = rT_ref[...]
    flow = flow_ref[...]

    q2 = jnp.sum(q * q, axis=1, keepdims=True)
    r2 = jnp.sum(rT * rT, axis=0, keepdims=True)
    qr = jnp.dot(q, rT, preferred_element_type=jnp.float32)
    d2 = q2 - 2.0 * qr + r2
    dist = jnp.sqrt(jnp.maximum(d2, 0.0))

    iota = jax.lax.broadcasted_iota(jnp.int32, dist.shape, 1)

    wsum = jnp.zeros((QB, 1), jnp.float32)
    acc = jnp.zeros((QB, DPAD), jnp.float32)
    for k in range(K):
        m = jnp.min(dist, axis=1, keepdims=True)
        idx = jnp.min(jnp.where(dist == m, iota, jnp.int32(2**30)),
                      axis=1, keepdims=True)
        sel = iota == idx
        onehot = sel.astype(jnp.float32)
        f = jnp.dot(onehot, flow, preferred_element_type=jnp.float32)
        w = 1.0 / (m + 1e-8)
        wsum = wsum + w
        acc = acc + w * f
        if k < K - 1:
            dist = jnp.where(sel, jnp.float32(jnp.inf), dist)

    out_ref[...] = acc / wsum


@jax.jit
def kernel(query_points, ref_points, ref_flow):
    n = query_points.shape[0]
    qp = jnp.zeros((n, DPAD), jnp.float32).at[:, :3].set(query_points)
    rT = jnp.zeros((DPAD, M), jnp.float32).at[:3, :].set(ref_points.T)
    fp = jnp.zeros((M, DPAD), jnp.float32).at[:, :3].set(ref_flow)

    grid = (n // QB,)
    out = pl.pallas_call(
        _body,
        grid=grid,
        in_specs=[
            pl.BlockSpec((QB, DPAD), lambda i: (i, 0)),
            pl.BlockSpec((DPAD, M), lambda i: (0, 0)),
            pl.BlockSpec((M, DPAD), lambda i: (0, 0)),
        ],
        out_specs=pl.BlockSpec((QB, DPAD), lambda i: (i, 0)),
        out_shape=jax.ShapeDtypeStruct((n, DPAD), jnp.float32),
    )(qp, rT, fp)
    return out[:, :3]
